# baseline (speedup 1.0000x reference)
import functools

import jax
import jax.numpy as jnp
from jax import lax
from jax.experimental import pallas as pl
from jax.experimental.pallas import tpu as pltpu
from jax.experimental.pallas import tpu_sc as plsc

NN = 10000
EE = 320000
DD = 128
NSTEPS = 6

NC = 2
NS = 16
KCH = 128
NCH = 80
EPT = NCH * KCH
EPAD = NC * NS * EPT
NACC = 10240
ZROWS = 640
EB = 8
NB = NCH // EB

BLK = 1000
GRID = NN // BLK


def _sc_scatter_body(t_hbm, e_hbm, p_hbm,
                     ebuf, rows0, rows1, acc, semE, sem0, sem1):
    c = lax.axis_index("c")
    s = lax.axis_index("s")
    tid = c * NS + s
    base = tid * NCH

    def _zrow(i, _):
        for j in range(DD // 16):
            rows0[i, pl.ds(16 * j, 16)] = jnp.zeros((16,), jnp.float32)
        return 0
    lax.fori_loop(0, KCH, _zrow, 0)
    for r in range(ZROWS // KCH):
        pltpu.sync_copy(rows0, acc.at[pl.ds(s * ZROWS + r * KCH, KCH)])

    pltpu.async_copy(e_hbm.at[pl.ds(base, EB)], ebuf.at[0], semE).wait()
    plsc.subcore_barrier()

    def _block(b, _):
        cur = lax.rem(b, 2)
        nxt = 1 - cur

        @pl.when(b + 1 < NB)
        def _():
            pltpu.async_copy(e_hbm.at[pl.ds(base + (b + 1) * EB, EB)],
                             ebuf.at[nxt], semE)

        eb = ebuf.at[cur]
        pltpu.async_copy(t_hbm.at[eb.at[0, 0]], rows0, sem0)
        for i in range(EB // 2):
            j0 = 2 * i
            pltpu.async_copy(t_hbm.at[eb.at[j0 + 1, 0]], rows1, sem1)
            pltpu.make_async_copy(t_hbm.at[eb.at[j0, 0]], rows0, sem0).wait()
            pltpu.sync_copy(rows0, acc.at[eb.at[j0, 1]], add=True)
            if j0 + 2 < EB:
                pltpu.async_copy(t_hbm.at[eb.at[j0 + 2, 0]], rows0, sem0)
            pltpu.make_async_copy(t_hbm.at[eb.at[j0 + 1, 0]], rows1,
                                  sem1).wait()
            pltpu.sync_copy(rows1, acc.at[eb.at[j0 + 1, 1]], add=True)

        @pl.when(b + 1 < NB)
        def _():
            pltpu.make_async_copy(e_hbm.at[pl.ds(base + (b + 1) * EB, EB)],
                                  ebuf.at[nxt], semE).wait()
        return 0
    lax.fori_loop(0, NB, _block, 0)

    plsc.subcore_barrier()
    pltpu.sync_copy(acc.at[pl.ds(s * ZROWS, ZROWS)],
                    p_hbm.at[c, pl.ds(s * ZROWS, ZROWS)])


def _sc_scatter(t, e):
    kern = pl.kernel(
        _sc_scatter_body,
        out_type=jax.ShapeDtypeStruct((NC, NACC, DD), jnp.float32),
        mesh=plsc.VectorSubcoreMesh(core_axis_name="c", subcore_axis_name="s",
                                    num_cores=NC, num_subcores=NS),
        scratch_types=[
            pltpu.VMEM((2, EB, 2, KCH), jnp.int32),
            pltpu.VMEM((KCH, DD), jnp.float32),
            pltpu.VMEM((KCH, DD), jnp.float32),
            pltpu.VMEM_SHARED((NACC, DD), jnp.float32),
            pltpu.SemaphoreType.DMA,
            pltpu.SemaphoreType.DMA,
            pltpu.SemaphoreType.DMA,
        ],
    )
    return kern(t, e)


def _mm_t(x, w):
    return lax.dot_general(x, w, (((1,), (1,)), ((), ())),
                           preferred_element_type=jnp.float32)


def _pre_body(nf, W0, b0, We, be, h_out, t_out):
    h = jnp.maximum(_mm_t(nf[...], W0[...]) + b0[...], 0.0)
    h_out[...] = h
    t_out[...] = _mm_t(h, We[...]) + be[...]


def _step_body(p, feat, t, Wih, bih, Whh, bhh, We, be, f_out, t_out):
    pr = p[...]
    ft = feat[...]
    a = pr[0] + pr[1] + t[...]
    gi = _mm_t(a, Wih[...]) + bih[...]
    gh = _mm_t(ft, Whh[...]) + bhh[...]
    r = jax.nn.sigmoid(gi[:, :DD] + gh[:, :DD])
    z = jax.nn.sigmoid(gi[:, DD:2 * DD] + gh[:, DD:2 * DD])
    nc = jnp.tanh(gi[:, 2 * DD:] + r * gh[:, 2 * DD:])
    f = (1.0 - z) * nc + z * ft
    f_out[...] = f
    t_out[...] = _mm_t(f, We[...]) + be[...]


def _post_body(feat, h, init, WmA, WmB, bm, out):
    out[...] = (_mm_t(feat[...], WmA[...]) + _mm_t(h[...], WmB[...])
                + bm[...] + init[...])


def _row_spec():
    return pl.BlockSpec((BLK, DD), lambda i: (i, 0))


def _full_spec(shape):
    return pl.BlockSpec(shape, lambda i: tuple(0 for _ in shape))


def _pre_call(nf, W0, b0, We, be):
    return pl.pallas_call(
        _pre_body,
        grid=(GRID,),
        in_specs=[_row_spec(), _full_spec((DD, DD)), _full_spec((1, DD)),
                  _full_spec((DD, DD)), _full_spec((1, DD))],
        out_specs=[_row_spec(), _row_spec()],
        out_shape=[jax.ShapeDtypeStruct((NN, DD), jnp.float32),
                   jax.ShapeDtypeStruct((NN, DD), jnp.float32)],
    )(nf, W0, b0, We, be)


def _step_call(p, feat, t, Wih, bih, Whh, bhh, We, be):
    return pl.pallas_call(
        _step_body,
        grid=(GRID,),
        in_specs=[pl.BlockSpec((NC, BLK, DD), lambda i: (0, i, 0)),
                  _row_spec(), _row_spec(),
                  _full_spec((3 * DD, DD)), _full_spec((1, 3 * DD)),
                  _full_spec((3 * DD, DD)), _full_spec((1, 3 * DD)),
                  _full_spec((DD, DD)), _full_spec((1, DD))],
        out_specs=[_row_spec(), _row_spec()],
        out_shape=[jax.ShapeDtypeStruct((NN, DD), jnp.float32),
                   jax.ShapeDtypeStruct((NN, DD), jnp.float32)],
    )(p, feat, t, Wih, bih, Whh, bhh, We, be)


def _post_call(feat, h, init, WmA, WmB, bm):
    return pl.pallas_call(
        _post_body,
        grid=(GRID,),
        in_specs=[_row_spec(), _row_spec(), _row_spec(),
                  _full_spec((DD, DD)), _full_spec((DD, DD)),
                  _full_spec((1, DD))],
        out_specs=_row_spec(),
        out_shape=jax.ShapeDtypeStruct((NN, DD), jnp.float32),
    )(feat, h, init, WmA, WmB, bm)


def kernel(n_feat, edge_index, W0, b0, We, be, W_ih, b_ih, W_hh, b_hh, Wm, bm):
    pad = EPAD - EE
    src = jnp.concatenate(
        [edge_index[0], jnp.zeros((pad,), jnp.int32)]).reshape(-1, 1, KCH)
    dst = jnp.concatenate(
        [edge_index[1], jnp.full((pad,), NN, jnp.int32)]).reshape(-1, 1, KCH)
    e = jnp.concatenate([src, dst], axis=1)

    b0r = b0.reshape(1, DD)
    ber = be.reshape(1, DD)
    bihr = b_ih.reshape(1, 3 * DD)
    bhhr = b_hh.reshape(1, 3 * DD)
    bmr = bm.reshape(1, DD)
    WmA = Wm[:, :DD]
    WmB = Wm[:, DD:]

    h, t = _pre_call(n_feat, W0, b0r, We, ber)
    feat = h
    for _ in range(NSTEPS):
        p = _sc_scatter(t, e)
        feat, t = _step_call(p, feat, t, W_ih, bihr, W_hh, bhhr, We, ber)
    return _post_call(feat, h, n_feat, WmA, WmB, bmr)


# speedup vs baseline: 1.2050x; 1.2050x over previous
import jax
import jax.numpy as jnp
from jax import lax
from jax.experimental import pallas as pl
from jax.experimental.pallas import tpu as pltpu
from jax.experimental.pallas import tpu_sc as plsc

NN = 10000
EE = 320000
DD = 128
NSTEPS = 6

NC = 2
NS = 16
KCH = 128
NCH0 = 95
NCH1 = 63
EPAD = NS * (NCH0 + NCH1) * KCH
NACC = 10240
ZROWS = 640
ZBUF = 128

BLK = 1000
GRID = NN // BLK


def _sc_scatter_body(t_hbm, src_hbm, dst_hbm, p_hbm,
                     sidx, didx, rows, zbuf, acc, sem):
    c = lax.axis_index("c")
    s = lax.axis_index("s")

    def _zrow(i, _):
        for j in range(DD // 16):
            zbuf[i, pl.ds(16 * j, 16)] = jnp.zeros((16,), jnp.float32)
        return 0
    lax.fori_loop(0, ZBUF, _zrow, 0)
    for r in range(ZROWS // ZBUF):
        pltpu.sync_copy(zbuf, acc.at[pl.ds(s * ZROWS + r * ZBUF, ZBUF)])
    plsc.subcore_barrier()

    nch = jnp.where(c == 0, NCH0, NCH1)
    base = jnp.where(c == 0, s * (NCH0 * KCH),
                     NS * (NCH0 * KCH) + s * (NCH1 * KCH))

    def _chunk(j, _):
        off = base + j * KCH
        pltpu.sync_copy(src_hbm.at[pl.ds(off, KCH)], sidx)
        pltpu.sync_copy(dst_hbm.at[pl.ds(off, KCH)], didx)
        pltpu.async_copy(t_hbm.at[sidx], rows, sem).wait()
        pltpu.sync_copy(rows, acc.at[didx], add=True)
        return 0
    lax.fori_loop(0, nch, _chunk, 0)

    plsc.subcore_barrier()
    pltpu.sync_copy(acc.at[pl.ds(s * ZROWS, ZROWS)],
                    p_hbm.at[c, pl.ds(s * ZROWS, ZROWS)])


def _sc_scatter(t, src, dst):
    kern = pl.kernel(
        _sc_scatter_body,
        out_type=jax.ShapeDtypeStruct((NC, NACC, DD), jnp.float32),
        mesh=plsc.VectorSubcoreMesh(core_axis_name="c", subcore_axis_name="s",
                                    num_cores=NC, num_subcores=NS),
        scratch_types=[
            pltpu.VMEM((KCH,), jnp.int32),
            pltpu.VMEM((KCH,), jnp.int32),
            pltpu.VMEM((KCH, DD), jnp.float32),
            pltpu.VMEM((ZBUF, DD), jnp.float32),
            pltpu.VMEM_SHARED((NACC, DD), jnp.float32),
            pltpu.SemaphoreType.DMA,
        ],
    )
    return kern(t, src, dst)


def _mm_t(x, w):
    return lax.dot_general(x, w, (((1,), (1,)), ((), ())),
                           preferred_element_type=jnp.float32)


def _pre_body(nf, W0, b0, We, be, h_out, t_out):
    h = jnp.maximum(_mm_t(nf[...], W0[...]) + b0[...], 0.0)
    h_out[...] = h
    t_out[...] = _mm_t(h, We[...]) + be[...]


def _step_body(p, feat, t, Wih, bih, Whh, bhh, We, be, f_out, t_out):
    pr = p[...]
    ft = feat[...]
    a = pr[0] + pr[1] + t[...]
    gi = _mm_t(a, Wih[...]) + bih[...]
    gh = _mm_t(ft, Whh[...]) + bhh[...]
    r = jax.nn.sigmoid(gi[:, :DD] + gh[:, :DD])
    z = jax.nn.sigmoid(gi[:, DD:2 * DD] + gh[:, DD:2 * DD])
    nc = jnp.tanh(gi[:, 2 * DD:] + r * gh[:, 2 * DD:])
    f = (1.0 - z) * nc + z * ft
    f_out[...] = f
    t_out[...] = _mm_t(f, We[...]) + be[...]


def _post_body(feat, h, init, WmA, WmB, bm, out):
    out[...] = (_mm_t(feat[...], WmA[...]) + _mm_t(h[...], WmB[...])
                + bm[...] + init[...])


def _row_spec():
    return pl.BlockSpec((BLK, DD), lambda i: (i, 0))


def _full_spec(shape):
    return pl.BlockSpec(shape, lambda i: tuple(0 for _ in shape))


def _pre_call(nf, W0, b0, We, be):
    return pl.pallas_call(
        _pre_body,
        grid=(GRID,),
        in_specs=[_row_spec(), _full_spec((DD, DD)), _full_spec((1, DD)),
                  _full_spec((DD, DD)), _full_spec((1, DD))],
        out_specs=[_row_spec(), _row_spec()],
        out_shape=[jax.ShapeDtypeStruct((NN, DD), jnp.float32),
                   jax.ShapeDtypeStruct((NN, DD), jnp.float32)],
    )(nf, W0, b0, We, be)


def _step_call(p, feat, t, Wih, bih, Whh, bhh, We, be):
    return pl.pallas_call(
        _step_body,
        grid=(GRID,),
        in_specs=[pl.BlockSpec((NC, BLK, DD), lambda i: (0, i, 0)),
                  _row_spec(), _row_spec(),
                  _full_spec((3 * DD, DD)), _full_spec((1, 3 * DD)),
                  _full_spec((3 * DD, DD)), _full_spec((1, 3 * DD)),
                  _full_spec((DD, DD)), _full_spec((1, DD))],
        out_specs=[_row_spec(), _row_spec()],
        out_shape=[jax.ShapeDtypeStruct((NN, DD), jnp.float32),
                   jax.ShapeDtypeStruct((NN, DD), jnp.float32)],
    )(p, feat, t, Wih, bih, Whh, bhh, We, be)


def _post_call(feat, h, init, WmA, WmB, bm):
    return pl.pallas_call(
        _post_body,
        grid=(GRID,),
        in_specs=[_row_spec(), _row_spec(), _row_spec(),
                  _full_spec((DD, DD)), _full_spec((DD, DD)),
                  _full_spec((1, DD))],
        out_specs=_row_spec(),
        out_shape=jax.ShapeDtypeStruct((NN, DD), jnp.float32),
    )(feat, h, init, WmA, WmB, bm)


def kernel(n_feat, edge_index, W0, b0, We, be, W_ih, b_ih, W_hh, b_hh, Wm, bm):
    pad = EPAD - EE
    src = jnp.concatenate([edge_index[0], jnp.zeros((pad,), jnp.int32)])
    dst = jnp.concatenate([edge_index[1], jnp.full((pad,), NN, jnp.int32)])

    b0r = b0.reshape(1, DD)
    ber = be.reshape(1, DD)
    bihr = b_ih.reshape(1, 3 * DD)
    bhhr = b_hh.reshape(1, 3 * DD)
    bmr = bm.reshape(1, DD)
    WmA = Wm[:, :DD]
    WmB = Wm[:, DD:]

    h, t = _pre_call(n_feat, W0, b0r, We, ber)
    feat = h
    for _ in range(NSTEPS):
        p = _sc_scatter(t, src, dst)
        feat, t = _step_call(p, feat, t, W_ih, bihr, W_hh, bhhr, We, ber)
    return _post_call(feat, h, n_feat, WmA, WmB, bmr)


# speedup vs baseline: 1.2387x; 1.0280x over previous
import jax
import jax.numpy as jnp
from jax import lax
from jax.experimental import pallas as pl
from jax.experimental.pallas import tpu as pltpu
from jax.experimental.pallas import tpu_sc as plsc

NN = 10000
EE = 320000
DD = 128
NSTEPS = 6

NC = 2
NS = 16
KCH = 128
NCH0 = 101
NCH1 = 57
EPAD = NS * (NCH0 + NCH1) * KCH
NACC = 10240
ZROWS = 640
ZBUF = 128

BLK = 1000
GRID = NN // BLK


def _sc_scatter_body(t_hbm, src_hbm, dst_hbm, p_hbm,
                     sidx, didx, rows, zbuf, acc, sem):
    c = lax.axis_index("c")
    s = lax.axis_index("s")

    def _zrow(i, _):
        for j in range(DD // 16):
            zbuf[i, pl.ds(16 * j, 16)] = jnp.zeros((16,), jnp.float32)
        return 0
    lax.fori_loop(0, ZBUF, _zrow, 0)
    for r in range(ZROWS // ZBUF):
        pltpu.sync_copy(zbuf, acc.at[pl.ds(s * ZROWS + r * ZBUF, ZBUF)])
    plsc.subcore_barrier()

    nch = jnp.where(c == 0, NCH0, NCH1)
    base = jnp.where(c == 0, s * (NCH0 * KCH),
                     NS * (NCH0 * KCH) + s * (NCH1 * KCH))

    def _chunk(j, _):
        off = base + j * KCH
        pltpu.sync_copy(src_hbm.at[pl.ds(off, KCH)], sidx)
        pltpu.sync_copy(dst_hbm.at[pl.ds(off, KCH)], didx)
        pltpu.async_copy(t_hbm.at[sidx], rows, sem).wait()
        pltpu.sync_copy(rows, acc.at[didx], add=True)
        return 0
    lax.fori_loop(0, nch, _chunk, 0)

    plsc.subcore_barrier()
    pltpu.sync_copy(acc.at[pl.ds(s * ZROWS, ZROWS)],
                    p_hbm.at[c, pl.ds(s * ZROWS, ZROWS)])


def _sc_scatter(t, src, dst):
    kern = pl.kernel(
        _sc_scatter_body,
        out_type=jax.ShapeDtypeStruct((NC, NACC, DD), jnp.float32),
        mesh=plsc.VectorSubcoreMesh(core_axis_name="c", subcore_axis_name="s",
                                    num_cores=NC, num_subcores=NS),
        scratch_types=[
            pltpu.VMEM((KCH,), jnp.int32),
            pltpu.VMEM((KCH,), jnp.int32),
            pltpu.VMEM((KCH, DD), jnp.float32),
            pltpu.VMEM((ZBUF, DD), jnp.float32),
            pltpu.VMEM_SHARED((NACC, DD), jnp.float32),
            pltpu.SemaphoreType.DMA,
        ],
    )
    return kern(t, src, dst)


def _mm_t(x, w):
    return lax.dot_general(x, w, (((1,), (1,)), ((), ())),
                           preferred_element_type=jnp.float32)


def _pre_body(nf, W0, b0, We, be, h_out, t_out):
    h = jnp.maximum(_mm_t(nf[...], W0[...]) + b0[...], 0.0)
    h_out[...] = h
    t_out[...] = _mm_t(h, We[...]) + be[...]


def _step_body(p, feat, t, Wih, bih, Whh, bhh, We, be, f_out, t_out):
    pr = p[...]
    ft = feat[...]
    a = pr[0] + pr[1] + t[...]
    gi = _mm_t(a, Wih[...]) + bih[...]
    gh = _mm_t(ft, Whh[...]) + bhh[...]
    r = jax.nn.sigmoid(gi[:, :DD] + gh[:, :DD])
    z = jax.nn.sigmoid(gi[:, DD:2 * DD] + gh[:, DD:2 * DD])
    nc = jnp.tanh(gi[:, 2 * DD:] + r * gh[:, 2 * DD:])
    f = (1.0 - z) * nc + z * ft
    f_out[...] = f
    t_out[...] = _mm_t(f, We[...]) + be[...]


def _post_body(feat, h, init, WmA, WmB, bm, out):
    out[...] = (_mm_t(feat[...], WmA[...]) + _mm_t(h[...], WmB[...])
                + bm[...] + init[...])


def _row_spec():
    return pl.BlockSpec((BLK, DD), lambda i: (i, 0))


def _full_spec(shape):
    return pl.BlockSpec(shape, lambda i: tuple(0 for _ in shape))


def _pre_call(nf, W0, b0, We, be):
    return pl.pallas_call(
        _pre_body,
        grid=(GRID,),
        in_specs=[_row_spec(), _full_spec((DD, DD)), _full_spec((1, DD)),
                  _full_spec((DD, DD)), _full_spec((1, DD))],
        out_specs=[_row_spec(), _row_spec()],
        out_shape=[jax.ShapeDtypeStruct((NN, DD), jnp.float32),
                   jax.ShapeDtypeStruct((NN, DD), jnp.float32)],
    )(nf, W0, b0, We, be)


def _step_call(p, feat, t, Wih, bih, Whh, bhh, We, be):
    return pl.pallas_call(
        _step_body,
        grid=(GRID,),
        in_specs=[pl.BlockSpec((NC, BLK, DD), lambda i: (0, i, 0)),
                  _row_spec(), _row_spec(),
                  _full_spec((3 * DD, DD)), _full_spec((1, 3 * DD)),
                  _full_spec((3 * DD, DD)), _full_spec((1, 3 * DD)),
                  _full_spec((DD, DD)), _full_spec((1, DD))],
        out_specs=[_row_spec(), _row_spec()],
        out_shape=[jax.ShapeDtypeStruct((NN, DD), jnp.float32),
                   jax.ShapeDtypeStruct((NN, DD), jnp.float32)],
    )(p, feat, t, Wih, bih, Whh, bhh, We, be)


def _post_call(feat, h, init, WmA, WmB, bm):
    return pl.pallas_call(
        _post_body,
        grid=(GRID,),
        in_specs=[_row_spec(), _row_spec(), _row_spec(),
                  _full_spec((DD, DD)), _full_spec((DD, DD)),
                  _full_spec((1, DD))],
        out_specs=_row_spec(),
        out_shape=jax.ShapeDtypeStruct((NN, DD), jnp.float32),
    )(feat, h, init, WmA, WmB, bm)


def kernel(n_feat, edge_index, W0, b0, We, be, W_ih, b_ih, W_hh, b_hh, Wm, bm):
    pad = EPAD - EE
    src = jnp.concatenate([edge_index[0], jnp.zeros((pad,), jnp.int32)])
    dst = jnp.concatenate([edge_index[1], jnp.full((pad,), NN, jnp.int32)])

    b0r = b0.reshape(1, DD)
    ber = be.reshape(1, DD)
    bihr = b_ih.reshape(1, 3 * DD)
    bhhr = b_hh.reshape(1, 3 * DD)
    bmr = bm.reshape(1, DD)
    WmA = Wm[:, :DD]
    WmB = Wm[:, DD:]

    h, t = _pre_call(n_feat, W0, b0r, We, ber)
    feat = h
    for _ in range(NSTEPS):
        p = _sc_scatter(t, src, dst)
        feat, t = _step_call(p, feat, t, W_ih, bihr, W_hh, bhhr, We, ber)
    return _post_call(feat, h, n_feat, WmA, WmB, bmr)


# speedup vs baseline: 1.3277x; 1.0718x over previous
import jax
import jax.numpy as jnp
from jax import lax
from jax.experimental import pallas as pl
from jax.experimental.pallas import tpu as pltpu
from jax.experimental.pallas import tpu_sc as plsc

NN = 10000
EE = 320000
DD = 128
NSTEPS = 6

NC = 2
NS = 16
KCH = 128
NCH0 = 101
NCH1 = 57
EPAD = NS * (NCH0 + NCH1) * KCH
NACC = 10240
ZROWS = 640
ZBUF = 128

BLK = 1000
GRID = NN // BLK


def _sc_scatter_body(t_hbm, e_hbm, p_hbm,
                     eidx, rows, zbuf, acc, sem):
    c = lax.axis_index("c")
    s = lax.axis_index("s")

    def _zrow(i, _):
        for j in range(DD // 16):
            zbuf[i, pl.ds(16 * j, 16)] = jnp.zeros((16,), jnp.float32)
        return 0
    lax.fori_loop(0, ZBUF, _zrow, 0)
    for r in range(ZROWS // ZBUF):
        pltpu.sync_copy(zbuf, acc.at[pl.ds(s * ZROWS + r * ZBUF, ZBUF)])
    plsc.subcore_barrier()

    nch = jnp.where(c == 0, NCH0, NCH1)
    base = jnp.where(c == 0, s * NCH0, NS * NCH0 + s * NCH1)

    def _chunk(j, _):
        pltpu.sync_copy(e_hbm.at[base + j], eidx)
        pltpu.async_copy(t_hbm.at[eidx.at[0]], rows, sem).wait()
        pltpu.sync_copy(rows, acc.at[eidx.at[1]], add=True)
        return 0
    lax.fori_loop(0, nch, _chunk, 0)

    plsc.subcore_barrier()
    pltpu.sync_copy(acc.at[pl.ds(s * ZROWS, ZROWS)],
                    p_hbm.at[c, pl.ds(s * ZROWS, ZROWS)])


def _sc_scatter(t, e):
    kern = pl.kernel(
        _sc_scatter_body,
        out_type=jax.ShapeDtypeStruct((NC, NACC, DD), jnp.float32),
        mesh=plsc.VectorSubcoreMesh(core_axis_name="c", subcore_axis_name="s",
                                    num_cores=NC, num_subcores=NS),
        scratch_types=[
            pltpu.VMEM((2, KCH), jnp.int32),
            pltpu.VMEM((KCH, DD), jnp.float32),
            pltpu.VMEM((ZBUF, DD), jnp.float32),
            pltpu.VMEM_SHARED((NACC, DD), jnp.float32),
            pltpu.SemaphoreType.DMA,
        ],
    )
    return kern(t, e)


def _mm_t(x, w):
    return lax.dot_general(x, w, (((1,), (1,)), ((), ())),
                           preferred_element_type=jnp.float32)


def _pre_body(nf, W0, b0, We, be, h_out, t_out):
    h = jnp.maximum(_mm_t(nf[...], W0[...]) + b0[...], 0.0)
    h_out[...] = h
    t_out[...] = _mm_t(h, We[...]) + be[...]


def _step_body(p, feat, t, Wih, bih, Whh, bhh, We, be, f_out, t_out):
    pr = p[...]
    ft = feat[...]
    a = pr[0] + pr[1] + t[...]
    gi = _mm_t(a, Wih[...]) + bih[...]
    gh = _mm_t(ft, Whh[...]) + bhh[...]
    r = jax.nn.sigmoid(gi[:, :DD] + gh[:, :DD])
    z = jax.nn.sigmoid(gi[:, DD:2 * DD] + gh[:, DD:2 * DD])
    nc = jnp.tanh(gi[:, 2 * DD:] + r * gh[:, 2 * DD:])
    f = (1.0 - z) * nc + z * ft
    f_out[...] = f
    t_out[...] = _mm_t(f, We[...]) + be[...]


def _post_body(feat, h, init, WmA, WmB, bm, out):
    out[...] = (_mm_t(feat[...], WmA[...]) + _mm_t(h[...], WmB[...])
                + bm[...] + init[...])


def _row_spec():
    return pl.BlockSpec((BLK, DD), lambda i: (i, 0))


def _full_spec(shape):
    return pl.BlockSpec(shape, lambda i: tuple(0 for _ in shape))


def _pre_call(nf, W0, b0, We, be):
    return pl.pallas_call(
        _pre_body,
        grid=(GRID,),
        in_specs=[_row_spec(), _full_spec((DD, DD)), _full_spec((1, DD)),
                  _full_spec((DD, DD)), _full_spec((1, DD))],
        out_specs=[_row_spec(), _row_spec()],
        out_shape=[jax.ShapeDtypeStruct((NN, DD), jnp.float32),
                   jax.ShapeDtypeStruct((NN, DD), jnp.float32)],
    )(nf, W0, b0, We, be)


def _step_call(p, feat, t, Wih, bih, Whh, bhh, We, be):
    return pl.pallas_call(
        _step_body,
        grid=(GRID,),
        in_specs=[pl.BlockSpec((NC, BLK, DD), lambda i: (0, i, 0)),
                  _row_spec(), _row_spec(),
                  _full_spec((3 * DD, DD)), _full_spec((1, 3 * DD)),
                  _full_spec((3 * DD, DD)), _full_spec((1, 3 * DD)),
                  _full_spec((DD, DD)), _full_spec((1, DD))],
        out_specs=[_row_spec(), _row_spec()],
        out_shape=[jax.ShapeDtypeStruct((NN, DD), jnp.float32),
                   jax.ShapeDtypeStruct((NN, DD), jnp.float32)],
    )(p, feat, t, Wih, bih, Whh, bhh, We, be)


def _post_call(feat, h, init, WmA, WmB, bm):
    return pl.pallas_call(
        _post_body,
        grid=(GRID,),
        in_specs=[_row_spec(), _row_spec(), _row_spec(),
                  _full_spec((DD, DD)), _full_spec((DD, DD)),
                  _full_spec((1, DD))],
        out_specs=_row_spec(),
        out_shape=jax.ShapeDtypeStruct((NN, DD), jnp.float32),
    )(feat, h, init, WmA, WmB, bm)


def kernel(n_feat, edge_index, W0, b0, We, be, W_ih, b_ih, W_hh, b_hh, Wm, bm):
    pad = EPAD - EE
    src = jnp.concatenate(
        [edge_index[0], jnp.zeros((pad,), jnp.int32)]).reshape(-1, 1, KCH)
    dst = jnp.concatenate(
        [edge_index[1], jnp.full((pad,), NN, jnp.int32)]).reshape(-1, 1, KCH)
    e = jnp.concatenate([src, dst], axis=1)

    b0r = b0.reshape(1, DD)
    ber = be.reshape(1, DD)
    bihr = b_ih.reshape(1, 3 * DD)
    bhhr = b_hh.reshape(1, 3 * DD)
    bmr = bm.reshape(1, DD)
    WmA = Wm[:, :DD]
    WmB = Wm[:, DD:]

    h, t = _pre_call(n_feat, W0, b0r, We, ber)
    feat = h
    for _ in range(NSTEPS):
        p = _sc_scatter(t, e)
        feat, t = _step_call(p, feat, t, W_ih, bihr, W_hh, bhhr, We, ber)
    return _post_call(feat, h, n_feat, WmA, WmB, bmr)


# speedup vs baseline: 1.5678x; 1.1809x over previous
import jax
import jax.numpy as jnp
from jax import lax
from jax.experimental import pallas as pl
from jax.experimental.pallas import tpu as pltpu
from jax.experimental.pallas import tpu_sc as plsc

NN = 10000
EE = 320000
DD = 128
NSTEPS = 6

NC = 2
NS = 16
KCH = 128
NCH0 = 102
NCH1 = 56
EPAD = NS * (NCH0 + NCH1) * KCH
NACC = 10240
ZROWS = 640
ZBUF = 128

BLK = 1000
GRID = NN // BLK


def _sc_scatter_body(t_hbm, e_hbm, p_hbm,
                     e0, e1, rows0, rows1, acc, semE, sem0, sem1):
    c = lax.axis_index("c")
    s = lax.axis_index("s")

    def _zrow(i, _):
        for j in range(DD // 16):
            rows0[i, pl.ds(16 * j, 16)] = jnp.zeros((16,), jnp.float32)
        return 0
    lax.fori_loop(0, ZBUF, _zrow, 0)
    for r in range(ZROWS // ZBUF):
        pltpu.sync_copy(rows0, acc.at[pl.ds(s * ZROWS + r * ZBUF, ZBUF)])
    plsc.subcore_barrier()

    nch = jnp.where(c == 0, NCH0, NCH1)
    base = jnp.where(c == 0, s * NCH0, NS * NCH0 + s * NCH1)

    pltpu.async_copy(e_hbm.at[base], e0, semE).wait()
    pltpu.async_copy(t_hbm.at[e0.at[0]], rows0, sem0)
    pltpu.async_copy(e_hbm.at[base + 1], e1, semE)

    def _pair(i, _):
        j0 = 2 * i
        pltpu.make_async_copy(e_hbm.at[base + j0 + 1], e1, semE).wait()
        pltpu.make_async_copy(t_hbm.at[e0.at[0]], rows0, sem0).wait()
        pltpu.async_copy(t_hbm.at[e1.at[0]], rows1, sem1)
        pltpu.sync_copy(rows0, acc.at[e0.at[1]], add=True)

        @pl.when(j0 + 2 < nch)
        def _():
            pltpu.async_copy(e_hbm.at[base + j0 + 2], e0, semE)
        pltpu.make_async_copy(t_hbm.at[e1.at[0]], rows1, sem1).wait()

        @pl.when(j0 + 2 < nch)
        def _():
            pltpu.make_async_copy(e_hbm.at[base + j0 + 2], e0, semE).wait()
            pltpu.async_copy(t_hbm.at[e0.at[0]], rows0, sem0)
        pltpu.sync_copy(rows1, acc.at[e1.at[1]], add=True)

        @pl.when(j0 + 3 < nch)
        def _():
            pltpu.async_copy(e_hbm.at[base + j0 + 3], e1, semE)
        return 0
    lax.fori_loop(0, nch // 2, _pair, 0)

    plsc.subcore_barrier()
    pltpu.sync_copy(acc.at[pl.ds(s * ZROWS, ZROWS)],
                    p_hbm.at[c, pl.ds(s * ZROWS, ZROWS)])


def _sc_scatter(t, e):
    kern = pl.kernel(
        _sc_scatter_body,
        out_type=jax.ShapeDtypeStruct((NC, NACC, DD), jnp.float32),
        mesh=plsc.VectorSubcoreMesh(core_axis_name="c", subcore_axis_name="s",
                                    num_cores=NC, num_subcores=NS),
        scratch_types=[
            pltpu.VMEM((2, KCH), jnp.int32),
            pltpu.VMEM((2, KCH), jnp.int32),
            pltpu.VMEM((KCH, DD), jnp.float32),
            pltpu.VMEM((KCH, DD), jnp.float32),
            pltpu.VMEM_SHARED((NACC, DD), jnp.float32),
            pltpu.SemaphoreType.DMA,
            pltpu.SemaphoreType.DMA,
            pltpu.SemaphoreType.DMA,
        ],
    )
    return kern(t, e)


def _mm_t(x, w):
    return lax.dot_general(x, w, (((1,), (1,)), ((), ())),
                           preferred_element_type=jnp.float32)


def _pre_body(nf, W0, b0, We, be, h_out, t_out):
    h = jnp.maximum(_mm_t(nf[...], W0[...]) + b0[...], 0.0)
    h_out[...] = h
    t_out[...] = _mm_t(h, We[...]) + be[...]


def _step_body(p, feat, t, Wih, bih, Whh, bhh, We, be, f_out, t_out):
    pr = p[...]
    ft = feat[...]
    a = pr[0] + pr[1] + t[...]
    gi = _mm_t(a, Wih[...]) + bih[...]
    gh = _mm_t(ft, Whh[...]) + bhh[...]
    r = jax.nn.sigmoid(gi[:, :DD] + gh[:, :DD])
    z = jax.nn.sigmoid(gi[:, DD:2 * DD] + gh[:, DD:2 * DD])
    nc = jnp.tanh(gi[:, 2 * DD:] + r * gh[:, 2 * DD:])
    f = (1.0 - z) * nc + z * ft
    f_out[...] = f
    t_out[...] = _mm_t(f, We[...]) + be[...]


def _post_body(feat, h, init, WmA, WmB, bm, out):
    out[...] = (_mm_t(feat[...], WmA[...]) + _mm_t(h[...], WmB[...])
                + bm[...] + init[...])


def _row_spec():
    return pl.BlockSpec((BLK, DD), lambda i: (i, 0))


def _full_spec(shape):
    return pl.BlockSpec(shape, lambda i: tuple(0 for _ in shape))


def _pre_call(nf, W0, b0, We, be):
    return pl.pallas_call(
        _pre_body,
        grid=(GRID,),
        in_specs=[_row_spec(), _full_spec((DD, DD)), _full_spec((1, DD)),
                  _full_spec((DD, DD)), _full_spec((1, DD))],
        out_specs=[_row_spec(), _row_spec()],
        out_shape=[jax.ShapeDtypeStruct((NN, DD), jnp.float32),
                   jax.ShapeDtypeStruct((NN, DD), jnp.float32)],
    )(nf, W0, b0, We, be)


def _step_call(p, feat, t, Wih, bih, Whh, bhh, We, be):
    return pl.pallas_call(
        _step_body,
        grid=(GRID,),
        in_specs=[pl.BlockSpec((NC, BLK, DD), lambda i: (0, i, 0)),
                  _row_spec(), _row_spec(),
                  _full_spec((3 * DD, DD)), _full_spec((1, 3 * DD)),
                  _full_spec((3 * DD, DD)), _full_spec((1, 3 * DD)),
                  _full_spec((DD, DD)), _full_spec((1, DD))],
        out_specs=[_row_spec(), _row_spec()],
        out_shape=[jax.ShapeDtypeStruct((NN, DD), jnp.float32),
                   jax.ShapeDtypeStruct((NN, DD), jnp.float32)],
    )(p, feat, t, Wih, bih, Whh, bhh, We, be)


def _post_call(feat, h, init, WmA, WmB, bm):
    return pl.pallas_call(
        _post_body,
        grid=(GRID,),
        in_specs=[_row_spec(), _row_spec(), _row_spec(),
                  _full_spec((DD, DD)), _full_spec((DD, DD)),
                  _full_spec((1, DD))],
        out_specs=_row_spec(),
        out_shape=jax.ShapeDtypeStruct((NN, DD), jnp.float32),
    )(feat, h, init, WmA, WmB, bm)


def kernel(n_feat, edge_index, W0, b0, We, be, W_ih, b_ih, W_hh, b_hh, Wm, bm):
    pad = EPAD - EE
    src = jnp.concatenate(
        [edge_index[0], jnp.zeros((pad,), jnp.int32)]).reshape(-1, 1, KCH)
    dst = jnp.concatenate(
        [edge_index[1], jnp.full((pad,), NN, jnp.int32)]).reshape(-1, 1, KCH)
    e = jnp.concatenate([src, dst], axis=1)

    b0r = b0.reshape(1, DD)
    ber = be.reshape(1, DD)
    bihr = b_ih.reshape(1, 3 * DD)
    bhhr = b_hh.reshape(1, 3 * DD)
    bmr = bm.reshape(1, DD)
    WmA = Wm[:, :DD]
    WmB = Wm[:, DD:]

    h, t = _pre_call(n_feat, W0, b0r, We, ber)
    feat = h
    for _ in range(NSTEPS):
        p = _sc_scatter(t, e)
        feat, t = _step_call(p, feat, t, W_ih, bihr, W_hh, bhhr, We, ber)
    return _post_call(feat, h, n_feat, WmA, WmB, bmr)


# speedup vs baseline: 1.6594x; 1.0584x over previous
import jax
import jax.numpy as jnp
from jax import lax
from jax.experimental import pallas as pl
from jax.experimental.pallas import tpu as pltpu
from jax.experimental.pallas import tpu_sc as plsc

NN = 10000
EE = 320000
DD = 128
NSTEPS = 6

NC = 2
NS = 16
KCH = 128
NCH0 = 116
NCH1 = 42
EPAD = NS * (NCH0 + NCH1) * KCH
NACC = 10240
ZROWS = 640
ZBUF = 128

BLK = 1000
GRID = NN // BLK


def _sc_scatter_body(t_hbm, e_hbm, p_hbm,
                     e0, e1, rows0, rows1, acc, semE, sem0, sem1):
    c = lax.axis_index("c")
    s = lax.axis_index("s")

    def _zrow(i, _):
        for j in range(DD // 16):
            rows0[i, pl.ds(16 * j, 16)] = jnp.zeros((16,), jnp.float32)
        return 0
    lax.fori_loop(0, ZBUF, _zrow, 0)
    for r in range(ZROWS // ZBUF):
        pltpu.sync_copy(rows0, acc.at[pl.ds(s * ZROWS + r * ZBUF, ZBUF)])
    plsc.subcore_barrier()

    nch = jnp.where(c == 0, NCH0, NCH1)
    base = jnp.where(c == 0, s * NCH0, NS * NCH0 + s * NCH1)

    pltpu.async_copy(e_hbm.at[base], e0, semE).wait()
    pltpu.async_copy(t_hbm.at[e0.at[0]], rows0, sem0)
    pltpu.async_copy(e_hbm.at[base + 1], e1, semE)

    def _pair(i, _):
        j0 = 2 * i
        pltpu.make_async_copy(e_hbm.at[base + j0 + 1], e1, semE).wait()
        pltpu.make_async_copy(t_hbm.at[e0.at[0]], rows0, sem0).wait()
        pltpu.async_copy(t_hbm.at[e1.at[0]], rows1, sem1)
        pltpu.sync_copy(rows0, acc.at[e0.at[1]], add=True)

        @pl.when(j0 + 2 < nch)
        def _():
            pltpu.async_copy(e_hbm.at[base + j0 + 2], e0, semE)
        pltpu.make_async_copy(t_hbm.at[e1.at[0]], rows1, sem1).wait()

        @pl.when(j0 + 2 < nch)
        def _():
            pltpu.make_async_copy(e_hbm.at[base + j0 + 2], e0, semE).wait()
            pltpu.async_copy(t_hbm.at[e0.at[0]], rows0, sem0)
        pltpu.sync_copy(rows1, acc.at[e1.at[1]], add=True)

        @pl.when(j0 + 3 < nch)
        def _():
            pltpu.async_copy(e_hbm.at[base + j0 + 3], e1, semE)
        return 0
    lax.fori_loop(0, nch // 2, _pair, 0)

    plsc.subcore_barrier()
    pltpu.sync_copy(acc.at[pl.ds(s * ZROWS, ZROWS)],
                    p_hbm.at[c, pl.ds(s * ZROWS, ZROWS)])


def _sc_scatter(t, e):
    kern = pl.kernel(
        _sc_scatter_body,
        out_type=jax.ShapeDtypeStruct((NC, NACC, DD), jnp.float32),
        mesh=plsc.VectorSubcoreMesh(core_axis_name="c", subcore_axis_name="s",
                                    num_cores=NC, num_subcores=NS),
        scratch_types=[
            pltpu.VMEM((2, KCH), jnp.int32),
            pltpu.VMEM((2, KCH), jnp.int32),
            pltpu.VMEM((KCH, DD), jnp.float32),
            pltpu.VMEM((KCH, DD), jnp.float32),
            pltpu.VMEM_SHARED((NACC, DD), jnp.float32),
            pltpu.SemaphoreType.DMA,
            pltpu.SemaphoreType.DMA,
            pltpu.SemaphoreType.DMA,
        ],
    )
    return kern(t, e)


def _mm_t(x, w):
    return lax.dot_general(x, w, (((1,), (1,)), ((), ())),
                           preferred_element_type=jnp.float32)


def _pre_body(nf, W0, b0, We, be, h_out, t_out):
    h = jnp.maximum(_mm_t(nf[...], W0[...]) + b0[...], 0.0)
    h_out[...] = h
    t_out[...] = _mm_t(h, We[...]) + be[...]


def _step_body(p, feat, t, Wih, bih, Whh, bhh, We, be, f_out, t_out):
    pr = p[...]
    ft = feat[...]
    a = pr[0] + pr[1] + t[...]
    gi = _mm_t(a, Wih[...]) + bih[...]
    gh = _mm_t(ft, Whh[...]) + bhh[...]
    r = jax.nn.sigmoid(gi[:, :DD] + gh[:, :DD])
    z = jax.nn.sigmoid(gi[:, DD:2 * DD] + gh[:, DD:2 * DD])
    nc = jnp.tanh(gi[:, 2 * DD:] + r * gh[:, 2 * DD:])
    f = (1.0 - z) * nc + z * ft
    f_out[...] = f
    t_out[...] = _mm_t(f, We[...]) + be[...]


def _post_body(feat, h, init, WmA, WmB, bm, out):
    out[...] = (_mm_t(feat[...], WmA[...]) + _mm_t(h[...], WmB[...])
                + bm[...] + init[...])


def _row_spec():
    return pl.BlockSpec((BLK, DD), lambda i: (i, 0))


def _full_spec(shape):
    return pl.BlockSpec(shape, lambda i: tuple(0 for _ in shape))


def _pre_call(nf, W0, b0, We, be):
    return pl.pallas_call(
        _pre_body,
        grid=(GRID,),
        in_specs=[_row_spec(), _full_spec((DD, DD)), _full_spec((1, DD)),
                  _full_spec((DD, DD)), _full_spec((1, DD))],
        out_specs=[_row_spec(), _row_spec()],
        out_shape=[jax.ShapeDtypeStruct((NN, DD), jnp.float32),
                   jax.ShapeDtypeStruct((NN, DD), jnp.float32)],
    )(nf, W0, b0, We, be)


def _step_call(p, feat, t, Wih, bih, Whh, bhh, We, be):
    return pl.pallas_call(
        _step_body,
        grid=(GRID,),
        in_specs=[pl.BlockSpec((NC, BLK, DD), lambda i: (0, i, 0)),
                  _row_spec(), _row_spec(),
                  _full_spec((3 * DD, DD)), _full_spec((1, 3 * DD)),
                  _full_spec((3 * DD, DD)), _full_spec((1, 3 * DD)),
                  _full_spec((DD, DD)), _full_spec((1, DD))],
        out_specs=[_row_spec(), _row_spec()],
        out_shape=[jax.ShapeDtypeStruct((NN, DD), jnp.float32),
                   jax.ShapeDtypeStruct((NN, DD), jnp.float32)],
    )(p, feat, t, Wih, bih, Whh, bhh, We, be)


def _post_call(feat, h, init, WmA, WmB, bm):
    return pl.pallas_call(
        _post_body,
        grid=(GRID,),
        in_specs=[_row_spec(), _row_spec(), _row_spec(),
                  _full_spec((DD, DD)), _full_spec((DD, DD)),
                  _full_spec((1, DD))],
        out_specs=_row_spec(),
        out_shape=jax.ShapeDtypeStruct((NN, DD), jnp.float32),
    )(feat, h, init, WmA, WmB, bm)


def kernel(n_feat, edge_index, W0, b0, We, be, W_ih, b_ih, W_hh, b_hh, Wm, bm):
    pad = EPAD - EE
    src = jnp.concatenate(
        [edge_index[0], jnp.zeros((pad,), jnp.int32)]).reshape(-1, 1, KCH)
    dst = jnp.concatenate(
        [edge_index[1], jnp.full((pad,), NN, jnp.int32)]).reshape(-1, 1, KCH)
    e = jnp.concatenate([src, dst], axis=1)

    b0r = b0.reshape(1, DD)
    ber = be.reshape(1, DD)
    bihr = b_ih.reshape(1, 3 * DD)
    bhhr = b_hh.reshape(1, 3 * DD)
    bmr = bm.reshape(1, DD)
    WmA = Wm[:, :DD]
    WmB = Wm[:, DD:]

    h, t = _pre_call(n_feat, W0, b0r, We, ber)
    feat = h
    for _ in range(NSTEPS):
        p = _sc_scatter(t, e)
        feat, t = _step_call(p, feat, t, W_ih, bihr, W_hh, bhhr, We, ber)
    return _post_call(feat, h, n_feat, WmA, WmB, bmr)
